# baseline (speedup 1.0000x reference)
import functools

import jax
import jax.numpy as jnp
from jax.experimental import pallas as pl

NUM_TOKENS = 16384
HIDDEN = 2048
EXPERTS = 64
BLOCK = 512
NSTREAM = 2
BAND = NUM_TOKENS // NSTREAM
STEPS = BAND // BLOCK


def _router_block(*refs):
    x_refs = refs[:NSTREAM]
    wt_ref = refs[NSTREAM]
    out_refs = refs[NSTREAM + 1:]
    wt = wt_ref[...]
    for j in range(NSTREAM):
        logits = jax.lax.dot_general(
            x_refs[j][...], wt, (((1,), (0,)), ((), ())),
            preferred_element_type=jnp.float32)
        m = jnp.max(logits, axis=1, keepdims=True)
        s = jnp.sum(jnp.exp(logits - m), axis=1, keepdims=True)
        lane = jax.lax.broadcasted_iota(jnp.int32, logits.shape, 1)
        idx = jnp.min(jnp.where(logits == m, lane, EXPERTS), axis=1,
                      keepdims=True)
        out_refs[2 * j][...] = 1.0 / s
        out_refs[2 * j + 1][...] = idx


@functools.partial(jax.jit, static_argnames=())
def kernel(hidden_states, W_gate):
    wt = W_gate.T
    x_specs = [
        pl.BlockSpec((BLOCK, HIDDEN),
                     functools.partial(lambda j, i: (i + j * STEPS, 0), j))
        for j in range(NSTREAM)
    ]
    out_specs = []
    out_shape = []
    for j in range(NSTREAM):
        out_specs += [pl.BlockSpec((BLOCK, 1), lambda i: (i, 0)),
                      pl.BlockSpec((BLOCK, 1), lambda i: (i, 0))]
        out_shape += [jax.ShapeDtypeStruct((BAND, 1), jnp.float32),
                      jax.ShapeDtypeStruct((BAND, 1), jnp.int32)]
    outs = pl.pallas_call(
        _router_block,
        grid=(STEPS,),
        in_specs=x_specs + [pl.BlockSpec((HIDDEN, EXPERTS), lambda i: (0, 0))],
        out_specs=out_specs,
        out_shape=out_shape,
    )(*([hidden_states] * NSTREAM + [wt]))
    weights = jnp.concatenate([outs[2 * j] for j in range(NSTREAM)], axis=0)
    indices = jnp.concatenate([outs[2 * j + 1] for j in range(NSTREAM)], axis=0)
    return weights, indices.astype(jnp.int64)


# speedup vs baseline: 1.3335x; 1.3335x over previous
import functools

import jax
import jax.numpy as jnp
from jax.experimental import pallas as pl

NUM_TOKENS = 16384
HIDDEN = 2048
EXPERTS = 64
BLOCK = 2048
STEPS = NUM_TOKENS // BLOCK
OROWS = BLOCK // 128


def _router_block(x_ref, wt_ref, w_out_ref, idx_out_ref):
    wt = wt_ref[...]
    logits = jax.lax.dot_general(
        x_ref[...], wt, (((1,), (0,)), ((), ())),
        preferred_element_type=jnp.float32)
    m = jnp.max(logits, axis=1, keepdims=True)
    s = jnp.sum(jnp.exp(logits - m), axis=1, keepdims=True)
    lane = jax.lax.broadcasted_iota(jnp.int32, logits.shape, 1)
    idx = jnp.min(jnp.where(logits == m, lane, EXPERTS), axis=1, keepdims=True)
    w_out_ref[...] = jnp.reshape(1.0 / s, (OROWS, 128))
    idx_out_ref[...] = jnp.reshape(idx, (OROWS, 128))


@functools.partial(jax.jit, static_argnames=())
def kernel(hidden_states, W_gate):
    wt = W_gate.T
    weights, indices = pl.pallas_call(
        _router_block,
        grid=(STEPS,),
        in_specs=[
            pl.BlockSpec((BLOCK, HIDDEN), lambda i: (i, 0)),
            pl.BlockSpec((HIDDEN, EXPERTS), lambda i: (0, 0)),
        ],
        out_specs=[
            pl.BlockSpec((OROWS, 128), lambda i: (i, 0)),
            pl.BlockSpec((OROWS, 128), lambda i: (i, 0)),
        ],
        out_shape=[
            jax.ShapeDtypeStruct((NUM_TOKENS // 128, 128), jnp.float32),
            jax.ShapeDtypeStruct((NUM_TOKENS // 128, 128), jnp.int32),
        ],
    )(hidden_states, wt)
    return (weights.reshape(NUM_TOKENS, 1),
            indices.reshape(NUM_TOKENS, 1).astype(jnp.int64))


# speedup vs baseline: 1.4070x; 1.0551x over previous
import functools

import jax
import jax.numpy as jnp
from jax.experimental import pallas as pl

NUM_TOKENS = 16384
HIDDEN = 2048
EXPERTS = 64
BLOCK = 2048
STEPS = NUM_TOKENS // BLOCK
OROWS = BLOCK // 128


def _router_block(x_ref, w_ref, w_out_ref, idx_out_ref):
    logits = jax.lax.dot_general(
        x_ref[...], w_ref[...], (((1,), (1,)), ((), ())),
        preferred_element_type=jnp.float32)
    m = jnp.max(logits, axis=1, keepdims=True)
    s = jnp.sum(jnp.exp(logits - m), axis=1, keepdims=True)
    lane = jax.lax.broadcasted_iota(jnp.int32, logits.shape, 1)
    idx = jnp.min(jnp.where(logits == m, lane, EXPERTS), axis=1, keepdims=True)
    w_out_ref[...] = jnp.reshape(1.0 / s, (OROWS, 128))
    idx_out_ref[...] = jnp.reshape(idx, (OROWS, 128))


@functools.partial(jax.jit, static_argnames=())
def kernel(hidden_states, W_gate):
    weights, indices = pl.pallas_call(
        _router_block,
        grid=(STEPS,),
        in_specs=[
            pl.BlockSpec((BLOCK, HIDDEN), lambda i: (i, 0)),
            pl.BlockSpec((EXPERTS, HIDDEN), lambda i: (0, 0)),
        ],
        out_specs=[
            pl.BlockSpec((OROWS, 128), lambda i: (i, 0)),
            pl.BlockSpec((OROWS, 128), lambda i: (i, 0)),
        ],
        out_shape=[
            jax.ShapeDtypeStruct((NUM_TOKENS // 128, 128), jnp.float32),
            jax.ShapeDtypeStruct((NUM_TOKENS // 128, 128), jnp.int32),
        ],
    )(hidden_states, W_gate)
    return (weights.reshape(NUM_TOKENS, 1),
            indices.reshape(NUM_TOKENS, 1).astype(jnp.int64))
